# baseline (speedup 1.0000x reference)
import jax
import jax.numpy as jnp
from jax.experimental import pallas as pl
from jax.experimental.pallas import tpu as pltpu

_BM = 640


def _fused_kernel(
    x_in_ref, adj_ref,
    m1w1_ref, m1b1_ref, m1w2_ref, m1b2_ref,
    m2w1_ref, m2b1_ref, m2w2_ref, m2b2_ref,
    wout_ref, wob_ref, wx_ref, wxb_ref, f2_w_ref, f2_b_ref,
    o_ref,
    x_scr,
):
    m = pl.program_id(0)

    @pl.when(m == 0)
    def _compute_x():
        h = jnp.maximum(
            jnp.dot(x_in_ref[...], m1w1_ref[...],
                    preferred_element_type=jnp.float32) + m1b1_ref[...],
            0.0,
        )
        x_scr[pl.ds(0, x_in_ref.shape[0]), :] = (
            jnp.dot(h, m1w2_ref[...], preferred_element_type=jnp.float32)
            + m1b2_ref[...]
        )

    nk = adj_ref.shape[1]
    out = jnp.dot(
        adj_ref[...].astype(jnp.bfloat16),
        x_scr[pl.ds(0, nk), :].astype(jnp.bfloat16),
        preferred_element_type=jnp.float32,
    )
    h = jnp.maximum(
        jnp.dot(out, m2w1_ref[...], preferred_element_type=jnp.float32)
        + m2b1_ref[...],
        0.0,
    )
    out = jnp.dot(h, m2w2_ref[...], preferred_element_type=jnp.float32) + m2b2_ref[...]
    x_blk = x_scr[pl.ds(m * _BM, _BM), :]
    d = o_ref.shape[1]
    t = (
        jnp.dot(out, wout_ref[...], preferred_element_type=jnp.float32)
        + wob_ref[...]
    )
    t2 = (
        jnp.dot(x_blk, wx_ref[...], preferred_element_type=jnp.float32)
        + wxb_ref[...]
    )
    z = jax.nn.sigmoid(t[:, :d] + t2[:, :d])
    r = jax.nn.sigmoid(t[:, d:2 * d] + t2[:, d:])
    out2 = jnp.tanh(
        t[:, 2 * d:]
        + jnp.dot(r * x_blk, f2_w_ref[...], preferred_element_type=jnp.float32)
        + f2_b_ref[...]
    )
    o_ref[...] = (1.0 - z) * x_blk + z * out2


def _full(shape):
    return pl.BlockSpec(shape, lambda m: (0, 0))


@jax.jit
def kernel(x_in, adj, mlp1_W1, mlp1_b1, mlp1_W2, mlp1_b2, mlp2_W1, mlp2_b1,
           mlp2_W2, mlp2_b2, fc1u_W, fc1u_b, fc2u_W, fc2u_b, fc1r_W, fc1r_b,
           fc2r_W, fc2r_b, fc1_W, fc1_b, fc2_W, fc2_b):
    n, din = x_in.shape
    dout = mlp1_W2.shape[1]
    biases = [mlp1_b1, mlp1_b2, mlp2_b1, mlp2_b2, fc1u_b, fc2u_b, fc1r_b,
              fc2r_b, fc1_b, fc2_b]
    (mlp1_b1, mlp1_b2, mlp2_b1, mlp2_b2, fc1u_b, fc2u_b, fc1r_b, fc2r_b,
     fc1_b, fc2_b) = [b.reshape(1, -1) for b in biases]

    w_out = jnp.concatenate([fc1u_W, fc1r_W, fc1_W], axis=1)
    b_out = jnp.concatenate([fc1u_b, fc1r_b, fc1_b], axis=1)
    w_x = jnp.concatenate([fc2u_W, fc2r_W], axis=1)
    b_x = jnp.concatenate([fc2u_b, fc2r_b], axis=1)
    nblocks = -(-n // _BM)
    grid = (nblocks,)
    w_spec = _full((din, dout))
    b_spec = _full((1, dout))
    out = pl.pallas_call(
        _fused_kernel,
        grid=grid,
        in_specs=[
            _full((n, din)),
            pl.BlockSpec((_BM, n), lambda m: (m, 0)),
            w_spec, b_spec, w_spec, b_spec,
            w_spec, b_spec, w_spec, b_spec,
            _full((din, 3 * dout)), _full((1, 3 * dout)),
            _full((din, 2 * dout)), _full((1, 2 * dout)),
            w_spec, b_spec,
        ],
        out_specs=pl.BlockSpec((_BM, dout), lambda m: (m, 0)),
        out_shape=jax.ShapeDtypeStruct((n, dout), jnp.float32),
        scratch_shapes=[
            pltpu.VMEM((nblocks * _BM, dout), jnp.float32),
        ],
        compiler_params=pltpu.CompilerParams(
            vmem_limit_bytes=64 * 1024 * 1024,
        ),
    )(x_in, adj, mlp1_W1, mlp1_b1, mlp1_W2, mlp1_b2, mlp2_W1, mlp2_b1,
      mlp2_W2, mlp2_b2, w_out, b_out, w_x, b_x, fc2_W, fc2_b)
    return out


# speedup vs baseline: 1.0455x; 1.0455x over previous
import jax
import jax.numpy as jnp
from jax.experimental import pallas as pl
from jax.experimental.pallas import tpu as pltpu

_BM = 640


def _fused_kernel(
    x_in_ref, adj_ref,
    m1w1_ref, m1b1_ref, m1w2_ref, m1b2_ref,
    m2w1_ref, m2b1_ref, m2w2_ref, m2b2_ref,
    f1u_w_ref, f1u_b_ref, f2u_w_ref, f2u_b_ref,
    f1r_w_ref, f1r_b_ref, f2r_w_ref, f2r_b_ref,
    f1_w_ref, f1_b_ref, f2_w_ref, f2_b_ref,
    o_ref,
    x_scr,
):
    m = pl.program_id(0)

    @pl.when(m == 0)
    def _compute_x():
        h = jnp.maximum(
            jnp.dot(x_in_ref[...], m1w1_ref[...],
                    preferred_element_type=jnp.float32) + m1b1_ref[...],
            0.0,
        )
        x_scr[pl.ds(0, x_in_ref.shape[0]), :] = (
            jnp.dot(h, m1w2_ref[...], preferred_element_type=jnp.float32)
            + m1b2_ref[...]
        )

    nk = adj_ref.shape[1]
    out = jnp.dot(
        adj_ref[...].astype(jnp.bfloat16),
        x_scr[pl.ds(0, nk), :].astype(jnp.bfloat16),
        preferred_element_type=jnp.float32,
    )
    h = jnp.maximum(
        jnp.dot(out, m2w1_ref[...], preferred_element_type=jnp.float32)
        + m2b1_ref[...],
        0.0,
    )
    out = jnp.dot(h, m2w2_ref[...], preferred_element_type=jnp.float32) + m2b2_ref[...]
    x_blk = x_scr[pl.ds(m * _BM, _BM), :]
    z = jax.nn.sigmoid(
        jnp.dot(out, f1u_w_ref[...], preferred_element_type=jnp.float32)
        + f1u_b_ref[...]
        + jnp.dot(x_blk, f2u_w_ref[...], preferred_element_type=jnp.float32)
        + f2u_b_ref[...]
    )
    r = jax.nn.sigmoid(
        jnp.dot(out, f1r_w_ref[...], preferred_element_type=jnp.float32)
        + f1r_b_ref[...]
        + jnp.dot(x_blk, f2r_w_ref[...], preferred_element_type=jnp.float32)
        + f2r_b_ref[...]
    )
    out2 = jnp.tanh(
        jnp.dot(out, f1_w_ref[...], preferred_element_type=jnp.float32)
        + f1_b_ref[...]
        + jnp.dot(r * x_blk, f2_w_ref[...], preferred_element_type=jnp.float32)
        + f2_b_ref[...]
    )
    o_ref[...] = (1.0 - z) * x_blk + z * out2


def _full(shape):
    return pl.BlockSpec(shape, lambda m: (0, 0))


@jax.jit
def kernel(x_in, adj, mlp1_W1, mlp1_b1, mlp1_W2, mlp1_b2, mlp2_W1, mlp2_b1,
           mlp2_W2, mlp2_b2, fc1u_W, fc1u_b, fc2u_W, fc2u_b, fc1r_W, fc1r_b,
           fc2r_W, fc2r_b, fc1_W, fc1_b, fc2_W, fc2_b):
    n, din = x_in.shape
    dout = mlp1_W2.shape[1]
    biases = [mlp1_b1, mlp1_b2, mlp2_b1, mlp2_b2, fc1u_b, fc2u_b, fc1r_b,
              fc2r_b, fc1_b, fc2_b]
    (mlp1_b1, mlp1_b2, mlp2_b1, mlp2_b2, fc1u_b, fc2u_b, fc1r_b, fc2r_b,
     fc1_b, fc2_b) = [b.reshape(1, -1) for b in biases]

    nblocks = -(-n // _BM)
    grid = (nblocks,)
    w_spec = _full((din, dout))
    b_spec = _full((1, dout))
    out = pl.pallas_call(
        _fused_kernel,
        grid=grid,
        in_specs=[
            _full((n, din)),
            pl.BlockSpec((_BM, n), lambda m: (m, 0)),
            w_spec, b_spec, w_spec, b_spec,
            w_spec, b_spec, w_spec, b_spec,
            w_spec, b_spec, w_spec, b_spec,
            w_spec, b_spec, w_spec, b_spec,
            w_spec, b_spec, w_spec, b_spec,
        ],
        out_specs=pl.BlockSpec((_BM, dout), lambda m: (m, 0)),
        out_shape=jax.ShapeDtypeStruct((n, dout), jnp.float32),
        scratch_shapes=[
            pltpu.VMEM((nblocks * _BM, dout), jnp.float32),
        ],
        compiler_params=pltpu.CompilerParams(
            vmem_limit_bytes=64 * 1024 * 1024,
        ),
    )(x_in, adj, mlp1_W1, mlp1_b1, mlp1_W2, mlp1_b2, mlp2_W1, mlp2_b1,
      mlp2_W2, mlp2_b2, fc1u_W, fc1u_b, fc2u_W, fc2u_b, fc1r_W, fc1r_b,
      fc2r_W, fc2r_b, fc1_W, fc1_b, fc2_W, fc2_b)
    return out


# speedup vs baseline: 1.0580x; 1.0120x over previous
import jax
import jax.numpy as jnp
from jax.experimental import pallas as pl
from jax.experimental.pallas import tpu as pltpu

_BM = 400


def _body(
    x_in_ref, adj_ref,
    m1w1_ref, m1b1_ref, m1w2_ref, m1b2_ref,
    m2w1_ref, m2b1_ref, m2w2_ref, m2b2_ref,
    f1u_w_ref, f1u_b_ref, f2u_w_ref, f2u_b_ref,
    f1r_w_ref, f1r_b_ref, f2r_w_ref, f2r_b_ref,
    f1_w_ref, f1_b_ref, f2_w_ref, f2_b_ref,
    o_ref,
    x_scr, abuf, res, in_sem, out_sem,
):
    m = pl.program_id(0)
    nb = pl.num_programs(0) - 1

    @pl.when(m < nb)
    def _start_fetch():
        pltpu.make_async_copy(
            adj_ref.at[pl.ds(m * _BM, _BM), :],
            abuf.at[m % 2],
            in_sem.at[m % 2],
        ).start()

    @pl.when(m == 0)
    def _compute_x():
        h = jnp.maximum(
            jnp.dot(x_in_ref[...], m1w1_ref[...],
                    preferred_element_type=jnp.float32) + m1b1_ref[...],
            0.0,
        )
        x_scr[...] = (
            jnp.dot(h, m1w2_ref[...], preferred_element_type=jnp.float32)
            + m1b2_ref[...]
        )

    @pl.when(m >= 1)
    def _compute_block():
        j = m - 1
        pltpu.make_async_copy(
            adj_ref.at[pl.ds(j * _BM, _BM), :],
            abuf.at[j % 2],
            in_sem.at[j % 2],
        ).wait()
        out = jnp.dot(
            abuf[j % 2].astype(jnp.bfloat16),
            x_scr[...].astype(jnp.bfloat16),
            preferred_element_type=jnp.float32,
        )
        h = jnp.maximum(
            jnp.dot(out, m2w1_ref[...], preferred_element_type=jnp.float32)
            + m2b1_ref[...],
            0.0,
        )
        out = (
            jnp.dot(h, m2w2_ref[...], preferred_element_type=jnp.float32)
            + m2b2_ref[...]
        )
        x_blk = x_scr[pl.ds(j * _BM, _BM), :]
        z = jax.nn.sigmoid(
            jnp.dot(out, f1u_w_ref[...], preferred_element_type=jnp.float32)
            + f1u_b_ref[...]
            + jnp.dot(x_blk, f2u_w_ref[...], preferred_element_type=jnp.float32)
            + f2u_b_ref[...]
        )
        r = jax.nn.sigmoid(
            jnp.dot(out, f1r_w_ref[...], preferred_element_type=jnp.float32)
            + f1r_b_ref[...]
            + jnp.dot(x_blk, f2r_w_ref[...], preferred_element_type=jnp.float32)
            + f2r_b_ref[...]
        )
        out2 = jnp.tanh(
            jnp.dot(out, f1_w_ref[...], preferred_element_type=jnp.float32)
            + f1_b_ref[...]
            + jnp.dot(r * x_blk, f2_w_ref[...], preferred_element_type=jnp.float32)
            + f2_b_ref[...]
        )

        @pl.when(m >= 3)
        def _drain_prev():
            pltpu.make_async_copy(
                res.at[j % 2],
                o_ref.at[pl.ds((j - 2) * _BM, _BM), :],
                out_sem.at[j % 2],
            ).wait()

        res[j % 2] = (1.0 - z) * x_blk + z * out2
        pltpu.make_async_copy(
            res.at[j % 2],
            o_ref.at[pl.ds(j * _BM, _BM), :],
            out_sem.at[j % 2],
        ).start()

    @pl.when(m == nb)
    def _drain_tail():
        j = m - 1
        pltpu.make_async_copy(
            res.at[(j - 1) % 2],
            o_ref.at[pl.ds((j - 1) * _BM, _BM), :],
            out_sem.at[(j - 1) % 2],
        ).wait()
        pltpu.make_async_copy(
            res.at[j % 2],
            o_ref.at[pl.ds(j * _BM, _BM), :],
            out_sem.at[j % 2],
        ).wait()


def _full(shape):
    return pl.BlockSpec(shape, lambda m: (0, 0))


@jax.jit
def kernel(x_in, adj, mlp1_W1, mlp1_b1, mlp1_W2, mlp1_b2, mlp2_W1, mlp2_b1,
           mlp2_W2, mlp2_b2, fc1u_W, fc1u_b, fc2u_W, fc2u_b, fc1r_W, fc1r_b,
           fc2r_W, fc2r_b, fc1_W, fc1_b, fc2_W, fc2_b):
    n, din = x_in.shape
    dout = mlp1_W2.shape[1]
    biases = [mlp1_b1, mlp1_b2, mlp2_b1, mlp2_b2, fc1u_b, fc2u_b, fc1r_b,
              fc2r_b, fc1_b, fc2_b]
    (mlp1_b1, mlp1_b2, mlp2_b1, mlp2_b2, fc1u_b, fc2u_b, fc1r_b, fc2r_b,
     fc1_b, fc2_b) = [b.reshape(1, -1) for b in biases]

    nb = n // _BM
    w_spec = _full((din, dout))
    b_spec = _full((1, dout))
    out = pl.pallas_call(
        _body,
        grid=(nb + 1,),
        in_specs=[
            _full((n, din)),
            pl.BlockSpec(memory_space=pl.ANY),
            w_spec, b_spec, w_spec, b_spec,
            w_spec, b_spec, w_spec, b_spec,
            w_spec, b_spec, w_spec, b_spec,
            w_spec, b_spec, w_spec, b_spec,
            w_spec, b_spec, w_spec, b_spec,
        ],
        out_specs=pl.BlockSpec(memory_space=pl.ANY),
        out_shape=jax.ShapeDtypeStruct((n, dout), jnp.float32),
        scratch_shapes=[
            pltpu.VMEM((n, dout), jnp.float32),
            pltpu.VMEM((2, _BM, n), jnp.float32),
            pltpu.VMEM((2, _BM, dout), jnp.float32),
            pltpu.SemaphoreType.DMA((2,)),
            pltpu.SemaphoreType.DMA((2,)),
        ],
        compiler_params=pltpu.CompilerParams(
            vmem_limit_bytes=64 * 1024 * 1024,
        ),
    )(x_in, adj, mlp1_W1, mlp1_b1, mlp1_W2, mlp1_b2, mlp2_W1, mlp2_b1,
      mlp2_W2, mlp2_b2, fc1u_W, fc1u_b, fc2u_W, fc2u_b, fc1r_W, fc1r_b,
      fc2r_W, fc2r_b, fc1_W, fc1_b, fc2_W, fc2_b)
    return out


# speedup vs baseline: 1.0589x; 1.0008x over previous
import jax
import jax.numpy as jnp
from jax.experimental import pallas as pl
from jax.experimental.pallas import tpu as pltpu

_BM = 400


def _body(
    x_in_ref, adj_ref,
    m1w1_ref, m1b1_ref, m1w2_ref, m1b2_ref,
    m2w1_ref, m2b1_ref, m2w2_ref, m2b2_ref,
    f1u_w_ref, f1u_b_ref, f2u_w_ref, f2u_b_ref,
    f1r_w_ref, f1r_b_ref, f2r_w_ref, f2r_b_ref,
    f1_w_ref, f1_b_ref, f2_w_ref, f2_b_ref,
    o_ref,
    x_scr, abuf, res, in_sem, out_sem,
):
    m = pl.program_id(0)
    nb = pl.num_programs(0) - 1

    _H = _BM // 2

    @pl.when(m < nb)
    def _start_fetch():
        pltpu.make_async_copy(
            adj_ref.at[pl.ds(m * _BM, _H), :],
            abuf.at[m % 2, pl.ds(0, _H), :],
            in_sem.at[m % 2, 0],
        ).start()
        pltpu.make_async_copy(
            adj_ref.at[pl.ds(m * _BM + _H, _H), :],
            abuf.at[m % 2, pl.ds(_H, _H), :],
            in_sem.at[m % 2, 1],
        ).start()

    @pl.when(m == 0)
    def _compute_x():
        h = jnp.maximum(
            jnp.dot(x_in_ref[...], m1w1_ref[...],
                    preferred_element_type=jnp.float32) + m1b1_ref[...],
            0.0,
        )
        x_scr[...] = (
            jnp.dot(h, m1w2_ref[...], preferred_element_type=jnp.float32)
            + m1b2_ref[...]
        )

    @pl.when(m >= 1)
    def _compute_block():
        j = m - 1
        pltpu.make_async_copy(
            adj_ref.at[pl.ds(j * _BM, _H), :],
            abuf.at[j % 2, pl.ds(0, _H), :],
            in_sem.at[j % 2, 0],
        ).wait()
        pltpu.make_async_copy(
            adj_ref.at[pl.ds(j * _BM + _H, _H), :],
            abuf.at[j % 2, pl.ds(_H, _H), :],
            in_sem.at[j % 2, 1],
        ).wait()
        out = jnp.dot(
            abuf[j % 2].astype(jnp.bfloat16),
            x_scr[...].astype(jnp.bfloat16),
            preferred_element_type=jnp.float32,
        )
        h = jnp.maximum(
            jnp.dot(out, m2w1_ref[...], preferred_element_type=jnp.float32)
            + m2b1_ref[...],
            0.0,
        )
        out = (
            jnp.dot(h, m2w2_ref[...], preferred_element_type=jnp.float32)
            + m2b2_ref[...]
        )
        x_blk = x_scr[pl.ds(j * _BM, _BM), :]
        z = jax.nn.sigmoid(
            jnp.dot(out, f1u_w_ref[...], preferred_element_type=jnp.float32)
            + f1u_b_ref[...]
            + jnp.dot(x_blk, f2u_w_ref[...], preferred_element_type=jnp.float32)
            + f2u_b_ref[...]
        )
        r = jax.nn.sigmoid(
            jnp.dot(out, f1r_w_ref[...], preferred_element_type=jnp.float32)
            + f1r_b_ref[...]
            + jnp.dot(x_blk, f2r_w_ref[...], preferred_element_type=jnp.float32)
            + f2r_b_ref[...]
        )
        out2 = jnp.tanh(
            jnp.dot(out, f1_w_ref[...], preferred_element_type=jnp.float32)
            + f1_b_ref[...]
            + jnp.dot(r * x_blk, f2_w_ref[...], preferred_element_type=jnp.float32)
            + f2_b_ref[...]
        )

        @pl.when(m >= 3)
        def _drain_prev():
            pltpu.make_async_copy(
                res.at[j % 2],
                o_ref.at[pl.ds((j - 2) * _BM, _BM), :],
                out_sem.at[j % 2],
            ).wait()

        res[j % 2] = (1.0 - z) * x_blk + z * out2
        pltpu.make_async_copy(
            res.at[j % 2],
            o_ref.at[pl.ds(j * _BM, _BM), :],
            out_sem.at[j % 2],
        ).start()

    @pl.when(m == nb)
    def _drain_tail():
        j = m - 1
        pltpu.make_async_copy(
            res.at[(j - 1) % 2],
            o_ref.at[pl.ds((j - 1) * _BM, _BM), :],
            out_sem.at[(j - 1) % 2],
        ).wait()
        pltpu.make_async_copy(
            res.at[j % 2],
            o_ref.at[pl.ds(j * _BM, _BM), :],
            out_sem.at[j % 2],
        ).wait()


def _full(shape):
    return pl.BlockSpec(shape, lambda m: (0, 0))


@jax.jit
def kernel(x_in, adj, mlp1_W1, mlp1_b1, mlp1_W2, mlp1_b2, mlp2_W1, mlp2_b1,
           mlp2_W2, mlp2_b2, fc1u_W, fc1u_b, fc2u_W, fc2u_b, fc1r_W, fc1r_b,
           fc2r_W, fc2r_b, fc1_W, fc1_b, fc2_W, fc2_b):
    n, din = x_in.shape
    dout = mlp1_W2.shape[1]
    biases = [mlp1_b1, mlp1_b2, mlp2_b1, mlp2_b2, fc1u_b, fc2u_b, fc1r_b,
              fc2r_b, fc1_b, fc2_b]
    (mlp1_b1, mlp1_b2, mlp2_b1, mlp2_b2, fc1u_b, fc2u_b, fc1r_b, fc2r_b,
     fc1_b, fc2_b) = [b.reshape(1, -1) for b in biases]

    nb = n // _BM
    w_spec = _full((din, dout))
    b_spec = _full((1, dout))
    out = pl.pallas_call(
        _body,
        grid=(nb + 1,),
        in_specs=[
            _full((n, din)),
            pl.BlockSpec(memory_space=pl.ANY),
            w_spec, b_spec, w_spec, b_spec,
            w_spec, b_spec, w_spec, b_spec,
            w_spec, b_spec, w_spec, b_spec,
            w_spec, b_spec, w_spec, b_spec,
            w_spec, b_spec, w_spec, b_spec,
        ],
        out_specs=pl.BlockSpec(memory_space=pl.ANY),
        out_shape=jax.ShapeDtypeStruct((n, dout), jnp.float32),
        scratch_shapes=[
            pltpu.VMEM((n, dout), jnp.float32),
            pltpu.VMEM((2, _BM, n), jnp.float32),
            pltpu.VMEM((2, _BM, dout), jnp.float32),
            pltpu.SemaphoreType.DMA((2, 2)),
            pltpu.SemaphoreType.DMA((2,)),
        ],
        compiler_params=pltpu.CompilerParams(
            vmem_limit_bytes=64 * 1024 * 1024,
        ),
    )(x_in, adj, mlp1_W1, mlp1_b1, mlp1_W2, mlp1_b2, mlp2_W1, mlp2_b1,
      mlp2_W2, mlp2_b2, fc1u_W, fc1u_b, fc2u_W, fc2u_b, fc1r_W, fc1r_b,
      fc2r_W, fc2r_b, fc1_W, fc1_b, fc2_W, fc2_b)
    return out


# speedup vs baseline: 1.0646x; 1.0054x over previous
import jax
import jax.numpy as jnp
from jax.experimental import pallas as pl
from jax.experimental.pallas import tpu as pltpu

_BM = 400


def _body(
    x_in_ref, adj_ref,
    m1w1_ref, m1b1_ref, m1w2_ref, m1b2_ref,
    m2w1_ref, m2b1_ref, m2w2_ref, m2b2_ref,
    f1u_w_ref, f1u_b_ref, f2u_w_ref, f2u_b_ref,
    f1r_w_ref, f1r_b_ref, f2r_w_ref, f2r_b_ref,
    f1_w_ref, f1_b_ref, f2_w_ref, f2_b_ref,
    o_ref,
    x_scr, abuf, res, in_sem, out_sem,
):
    m = pl.program_id(0)
    nb = pl.num_programs(0) - 1

    _H = _BM // 2

    @pl.when(m < nb)
    def _start_fetch():
        pltpu.make_async_copy(
            adj_ref.at[pl.ds(m * _BM, _H), :],
            abuf.at[m % 2, pl.ds(0, _H), :],
            in_sem.at[m % 2, 0],
        ).start()
        pltpu.make_async_copy(
            adj_ref.at[pl.ds(m * _BM + _H, _H), :],
            abuf.at[m % 2, pl.ds(_H, _H), :],
            in_sem.at[m % 2, 1],
        ).start()

    @pl.when(m == 0)
    def _compute_x():
        h = jnp.maximum(
            jnp.dot(x_in_ref[...], m1w1_ref[...],
                    preferred_element_type=jnp.float32) + m1b1_ref[...],
            0.0,
        )
        x_scr[...] = (
            jnp.dot(h, m1w2_ref[...], preferred_element_type=jnp.float32)
            + m1b2_ref[...]
        )

    def _process(a_blk, x_blk):
        out = jnp.dot(
            a_blk.astype(jnp.bfloat16),
            x_scr[...].astype(jnp.bfloat16),
            preferred_element_type=jnp.float32,
        )
        h = jnp.maximum(
            jnp.dot(out, m2w1_ref[...], preferred_element_type=jnp.float32)
            + m2b1_ref[...],
            0.0,
        )
        out = (
            jnp.dot(h, m2w2_ref[...], preferred_element_type=jnp.float32)
            + m2b2_ref[...]
        )
        z = jax.nn.sigmoid(
            jnp.dot(out, f1u_w_ref[...], preferred_element_type=jnp.float32)
            + f1u_b_ref[...]
            + jnp.dot(x_blk, f2u_w_ref[...], preferred_element_type=jnp.float32)
            + f2u_b_ref[...]
        )
        r = jax.nn.sigmoid(
            jnp.dot(out, f1r_w_ref[...], preferred_element_type=jnp.float32)
            + f1r_b_ref[...]
            + jnp.dot(x_blk, f2r_w_ref[...], preferred_element_type=jnp.float32)
            + f2r_b_ref[...]
        )
        out2 = jnp.tanh(
            jnp.dot(out, f1_w_ref[...], preferred_element_type=jnp.float32)
            + f1_b_ref[...]
            + jnp.dot(r * x_blk, f2_w_ref[...], preferred_element_type=jnp.float32)
            + f2_b_ref[...]
        )
        return (1.0 - z) * x_blk + z * out2

    @pl.when(m >= 1)
    def _compute_block():
        j = m - 1

        @pl.when(m >= 3)
        def _drain_prev():
            pltpu.make_async_copy(
                res.at[j % 2],
                o_ref.at[pl.ds((j - 2) * _BM, _BM), :],
                out_sem.at[j % 2],
            ).wait()

        @pl.when(j < nb - 1)
        def _whole_block():
            pltpu.make_async_copy(
                adj_ref.at[pl.ds(j * _BM, _H), :],
                abuf.at[j % 2, pl.ds(0, _H), :],
                in_sem.at[j % 2, 0],
            ).wait()
            pltpu.make_async_copy(
                adj_ref.at[pl.ds(j * _BM + _H, _H), :],
                abuf.at[j % 2, pl.ds(_H, _H), :],
                in_sem.at[j % 2, 1],
            ).wait()
            res[j % 2] = _process(
                abuf[j % 2], x_scr[pl.ds(j * _BM, _BM), :]
            )

        @pl.when(j == nb - 1)
        def _last_block_by_halves():
            pltpu.make_async_copy(
                adj_ref.at[pl.ds(j * _BM, _H), :],
                abuf.at[j % 2, pl.ds(0, _H), :],
                in_sem.at[j % 2, 0],
            ).wait()
            res[j % 2, pl.ds(0, _H), :] = _process(
                abuf[j % 2, pl.ds(0, _H), :],
                x_scr[pl.ds(j * _BM, _H), :],
            )
            pltpu.make_async_copy(
                adj_ref.at[pl.ds(j * _BM + _H, _H), :],
                abuf.at[j % 2, pl.ds(_H, _H), :],
                in_sem.at[j % 2, 1],
            ).wait()
            res[j % 2, pl.ds(_H, _H), :] = _process(
                abuf[j % 2, pl.ds(_H, _H), :],
                x_scr[pl.ds(j * _BM + _H, _H), :],
            )

        pltpu.make_async_copy(
            res.at[j % 2],
            o_ref.at[pl.ds(j * _BM, _BM), :],
            out_sem.at[j % 2],
        ).start()

    @pl.when(m == nb)
    def _drain_tail():
        j = m - 1
        pltpu.make_async_copy(
            res.at[(j - 1) % 2],
            o_ref.at[pl.ds((j - 1) * _BM, _BM), :],
            out_sem.at[(j - 1) % 2],
        ).wait()
        pltpu.make_async_copy(
            res.at[j % 2],
            o_ref.at[pl.ds(j * _BM, _BM), :],
            out_sem.at[j % 2],
        ).wait()


def _full(shape):
    return pl.BlockSpec(shape, lambda m: (0, 0))


@jax.jit
def kernel(x_in, adj, mlp1_W1, mlp1_b1, mlp1_W2, mlp1_b2, mlp2_W1, mlp2_b1,
           mlp2_W2, mlp2_b2, fc1u_W, fc1u_b, fc2u_W, fc2u_b, fc1r_W, fc1r_b,
           fc2r_W, fc2r_b, fc1_W, fc1_b, fc2_W, fc2_b):
    n, din = x_in.shape
    dout = mlp1_W2.shape[1]
    biases = [mlp1_b1, mlp1_b2, mlp2_b1, mlp2_b2, fc1u_b, fc2u_b, fc1r_b,
              fc2r_b, fc1_b, fc2_b]
    (mlp1_b1, mlp1_b2, mlp2_b1, mlp2_b2, fc1u_b, fc2u_b, fc1r_b, fc2r_b,
     fc1_b, fc2_b) = [b.reshape(1, -1) for b in biases]

    nb = n // _BM
    w_spec = _full((din, dout))
    b_spec = _full((1, dout))
    out = pl.pallas_call(
        _body,
        grid=(nb + 1,),
        in_specs=[
            _full((n, din)),
            pl.BlockSpec(memory_space=pl.ANY),
            w_spec, b_spec, w_spec, b_spec,
            w_spec, b_spec, w_spec, b_spec,
            w_spec, b_spec, w_spec, b_spec,
            w_spec, b_spec, w_spec, b_spec,
            w_spec, b_spec, w_spec, b_spec,
        ],
        out_specs=pl.BlockSpec(memory_space=pl.ANY),
        out_shape=jax.ShapeDtypeStruct((n, dout), jnp.float32),
        scratch_shapes=[
            pltpu.VMEM((n, dout), jnp.float32),
            pltpu.VMEM((2, _BM, n), jnp.float32),
            pltpu.VMEM((2, _BM, dout), jnp.float32),
            pltpu.SemaphoreType.DMA((2, 2)),
            pltpu.SemaphoreType.DMA((2,)),
        ],
        compiler_params=pltpu.CompilerParams(
            vmem_limit_bytes=64 * 1024 * 1024,
        ),
    )(x_in, adj, mlp1_W1, mlp1_b1, mlp1_W2, mlp1_b2, mlp2_W1, mlp2_b1,
      mlp2_W2, mlp2_b2, fc1u_W, fc1u_b, fc2u_W, fc2u_b, fc1r_W, fc1r_b,
      fc2r_W, fc2r_b, fc1_W, fc1_b, fc2_W, fc2_b)
    return out


# speedup vs baseline: 1.0706x; 1.0056x over previous
import jax
import jax.numpy as jnp
from jax.experimental import pallas as pl
from jax.experimental.pallas import tpu as pltpu

_BM = 400


def _body(
    x_in_ref, adj_ref,
    m1w1_ref, m1b1_ref, m1w2_ref, m1b2_ref,
    m2w1_ref, m2b1_ref, m2w2_ref, m2b2_ref,
    f1u_w_ref, f1u_b_ref, f2u_w_ref, f2u_b_ref,
    f1r_w_ref, f1r_b_ref, f2r_w_ref, f2r_b_ref,
    f1_w_ref, f1_b_ref, f2_w_ref, f2_b_ref,
    o_ref,
    x_scr, abuf, res, in_sem, out_sem,
):
    m = pl.program_id(0)
    nb = pl.num_programs(0) - 1

    _H = _BM // 2

    def _start_fetch_block(b):
        pltpu.make_async_copy(
            adj_ref.at[pl.ds(b * _BM, _H), :],
            abuf.at[b % 2, pl.ds(0, _H), :],
            in_sem.at[b % 2, 0],
        ).start()
        pltpu.make_async_copy(
            adj_ref.at[pl.ds(b * _BM + _H, _H), :],
            abuf.at[b % 2, pl.ds(_H, _H), :],
            in_sem.at[b % 2, 1],
        ).start()

    @pl.when(m == 0)
    def _start_first_fetches():
        _start_fetch_block(0)
        _start_fetch_block(1)

    @pl.when(m == 0)
    def _compute_x():
        h = jnp.maximum(
            jnp.dot(x_in_ref[...], m1w1_ref[...],
                    preferred_element_type=jnp.float32) + m1b1_ref[...],
            0.0,
        )
        x_scr[...] = (
            jnp.dot(h, m1w2_ref[...], preferred_element_type=jnp.float32)
            + m1b2_ref[...]
        )

    def _process(a_blk, x_blk):
        out = jnp.dot(
            a_blk.astype(jnp.bfloat16),
            x_scr[...].astype(jnp.bfloat16),
            preferred_element_type=jnp.float32,
        )
        h = jnp.maximum(
            jnp.dot(out, m2w1_ref[...], preferred_element_type=jnp.float32)
            + m2b1_ref[...],
            0.0,
        )
        out = (
            jnp.dot(h, m2w2_ref[...], preferred_element_type=jnp.float32)
            + m2b2_ref[...]
        )
        z = jax.nn.sigmoid(
            jnp.dot(out, f1u_w_ref[...], preferred_element_type=jnp.float32)
            + f1u_b_ref[...]
            + jnp.dot(x_blk, f2u_w_ref[...], preferred_element_type=jnp.float32)
            + f2u_b_ref[...]
        )
        r = jax.nn.sigmoid(
            jnp.dot(out, f1r_w_ref[...], preferred_element_type=jnp.float32)
            + f1r_b_ref[...]
            + jnp.dot(x_blk, f2r_w_ref[...], preferred_element_type=jnp.float32)
            + f2r_b_ref[...]
        )
        out2 = jnp.tanh(
            jnp.dot(out, f1_w_ref[...], preferred_element_type=jnp.float32)
            + f1_b_ref[...]
            + jnp.dot(r * x_blk, f2_w_ref[...], preferred_element_type=jnp.float32)
            + f2_b_ref[...]
        )
        return (1.0 - z) * x_blk + z * out2

    @pl.when(m >= 1)
    def _compute_block():
        j = m - 1

        @pl.when(m >= 3)
        def _drain_prev():
            pltpu.make_async_copy(
                res.at[j % 2],
                o_ref.at[pl.ds((j - 2) * _BM, _BM), :],
                out_sem.at[j % 2],
            ).wait()

        @pl.when(j < nb - 1)
        def _whole_block():
            pltpu.make_async_copy(
                adj_ref.at[pl.ds(j * _BM, _H), :],
                abuf.at[j % 2, pl.ds(0, _H), :],
                in_sem.at[j % 2, 0],
            ).wait()
            pltpu.make_async_copy(
                adj_ref.at[pl.ds(j * _BM + _H, _H), :],
                abuf.at[j % 2, pl.ds(_H, _H), :],
                in_sem.at[j % 2, 1],
            ).wait()
            res[j % 2] = _process(
                abuf[j % 2], x_scr[pl.ds(j * _BM, _BM), :]
            )

        @pl.when(j == nb - 1)
        def _last_block_by_halves():
            pltpu.make_async_copy(
                adj_ref.at[pl.ds(j * _BM, _H), :],
                abuf.at[j % 2, pl.ds(0, _H), :],
                in_sem.at[j % 2, 0],
            ).wait()
            res[j % 2, pl.ds(0, _H), :] = _process(
                abuf[j % 2, pl.ds(0, _H), :],
                x_scr[pl.ds(j * _BM, _H), :],
            )
            pltpu.make_async_copy(
                adj_ref.at[pl.ds(j * _BM + _H, _H), :],
                abuf.at[j % 2, pl.ds(_H, _H), :],
                in_sem.at[j % 2, 1],
            ).wait()
            res[j % 2, pl.ds(_H, _H), :] = _process(
                abuf[j % 2, pl.ds(_H, _H), :],
                x_scr[pl.ds(j * _BM + _H, _H), :],
            )

        pltpu.make_async_copy(
            res.at[j % 2],
            o_ref.at[pl.ds(j * _BM, _BM), :],
            out_sem.at[j % 2],
        ).start()

        @pl.when(j + 2 < nb)
        def _start_next_fetch():
            _start_fetch_block(j + 2)

    @pl.when(m == nb)
    def _drain_tail():
        j = m - 1
        pltpu.make_async_copy(
            res.at[(j - 1) % 2],
            o_ref.at[pl.ds((j - 1) * _BM, _BM), :],
            out_sem.at[(j - 1) % 2],
        ).wait()
        pltpu.make_async_copy(
            res.at[j % 2],
            o_ref.at[pl.ds(j * _BM, _BM), :],
            out_sem.at[j % 2],
        ).wait()


def _full(shape):
    return pl.BlockSpec(shape, lambda m: (0, 0))


@jax.jit
def kernel(x_in, adj, mlp1_W1, mlp1_b1, mlp1_W2, mlp1_b2, mlp2_W1, mlp2_b1,
           mlp2_W2, mlp2_b2, fc1u_W, fc1u_b, fc2u_W, fc2u_b, fc1r_W, fc1r_b,
           fc2r_W, fc2r_b, fc1_W, fc1_b, fc2_W, fc2_b):
    n, din = x_in.shape
    dout = mlp1_W2.shape[1]
    biases = [mlp1_b1, mlp1_b2, mlp2_b1, mlp2_b2, fc1u_b, fc2u_b, fc1r_b,
              fc2r_b, fc1_b, fc2_b]
    (mlp1_b1, mlp1_b2, mlp2_b1, mlp2_b2, fc1u_b, fc2u_b, fc1r_b, fc2r_b,
     fc1_b, fc2_b) = [b.reshape(1, -1) for b in biases]

    nb = n // _BM
    w_spec = _full((din, dout))
    b_spec = _full((1, dout))
    out = pl.pallas_call(
        _body,
        grid=(nb + 1,),
        in_specs=[
            _full((n, din)),
            pl.BlockSpec(memory_space=pl.ANY),
            w_spec, b_spec, w_spec, b_spec,
            w_spec, b_spec, w_spec, b_spec,
            w_spec, b_spec, w_spec, b_spec,
            w_spec, b_spec, w_spec, b_spec,
            w_spec, b_spec, w_spec, b_spec,
        ],
        out_specs=pl.BlockSpec(memory_space=pl.ANY),
        out_shape=jax.ShapeDtypeStruct((n, dout), jnp.float32),
        scratch_shapes=[
            pltpu.VMEM((n, dout), jnp.float32),
            pltpu.VMEM((2, _BM, n), jnp.float32),
            pltpu.VMEM((2, _BM, dout), jnp.float32),
            pltpu.SemaphoreType.DMA((2, 2)),
            pltpu.SemaphoreType.DMA((2,)),
        ],
        compiler_params=pltpu.CompilerParams(
            vmem_limit_bytes=64 * 1024 * 1024,
        ),
    )(x_in, adj, mlp1_W1, mlp1_b1, mlp1_W2, mlp1_b2, mlp2_W1, mlp2_b1,
      mlp2_W2, mlp2_b2, fc1u_W, fc1u_b, fc2u_W, fc2u_b, fc1r_W, fc1r_b,
      fc2r_W, fc2r_b, fc1_W, fc1_b, fc2_W, fc2_b)
    return out
